# Initial kernel scaffold; baseline (speedup 1.0000x reference)
import jax
import jax.numpy as jnp
from jax.experimental import pallas as pl


def kernel(x, edge_index, W, b, eps):
    raise NotImplementedError("write your pallas kernel here")


# speedup vs baseline: 7.9573x; 7.9573x over previous
import functools

import jax
import jax.numpy as jnp
from jax import lax
from jax.experimental import pallas as pl
from jax.experimental.pallas import tpu as pltpu
from jax.experimental.pallas import tpu_sc as plsc

N_NODES = 10000
N_PAD = 10240
D = 128
N_EDGES = 320000
NC = 2
NS = 16
NW = NC * NS
B = 128
CH = 79
E_PAD = NW * CH * B
ROWS_PER_TILE = N_PAD // NS


def _relu_body(x_ref, o_ref):
    o_ref[...] = jnp.maximum(x_ref[...], 0.0)


def _relu_table(x_pad):
    grid = N_PAD // 1024
    return pl.pallas_call(
        _relu_body,
        grid=(grid,),
        in_specs=[pl.BlockSpec((1024, D), lambda i: (i, 0))],
        out_specs=pl.BlockSpec((1024, D), lambda i: (i, 0)),
        out_shape=jax.ShapeDtypeStruct((N_PAD, D), jnp.float32),
    )(x_pad)


def _sc_agg_body(relux_hbm, src_hbm, dst_hbm, zeros_hbm, out_hbm,
                 src_v, dst_v, rows_v, agg_sh, sem):
    c = lax.axis_index("c")
    s = lax.axis_index("s")
    wid = s * NC + c

    pltpu.sync_copy(zeros_hbm, rows_v)
    for z in range(ROWS_PER_TILE // B):
        pltpu.sync_copy(rows_v, agg_sh.at[pl.ds(s * ROWS_PER_TILE + z * B, B)])

    pltpu.sync_copy(src_hbm.at[wid], src_v)
    pltpu.sync_copy(dst_hbm.at[wid], dst_v)
    plsc.subcore_barrier()

    def chunk(j, carry):
        pltpu.async_copy(relux_hbm.at[src_v.at[j]], rows_v, sem).wait()
        pltpu.sync_copy(rows_v, agg_sh.at[dst_v.at[j]], add=True)
        return carry

    lax.fori_loop(0, CH, chunk, 0)
    plsc.subcore_barrier()

    for z in range(ROWS_PER_TILE // B):
        r0 = s * ROWS_PER_TILE + z * B
        pltpu.sync_copy(agg_sh.at[pl.ds(r0, B)], rows_v)
        pltpu.sync_copy(rows_v, out_hbm.at[c].at[pl.ds(r0, B)])


_sc_agg = functools.partial(
    pl.kernel,
    out_type=jax.ShapeDtypeStruct((NC, N_PAD, D), jnp.float32),
    mesh=plsc.VectorSubcoreMesh(core_axis_name="c", subcore_axis_name="s"),
    scratch_types=[
        pltpu.VMEM((CH, B), jnp.int32),
        pltpu.VMEM((CH, B), jnp.int32),
        pltpu.VMEM((B, D), jnp.float32),
        pltpu.VMEM_SHARED((N_PAD, D), jnp.float32),
        pltpu.SemaphoreType.DMA,
    ],
)(_sc_agg_body)


def _final_body(scale_ref, x_ref, a0_ref, a1_ref, w_ref, b_ref, o_ref):
    h = scale_ref[0, 0] * x_ref[...] + a0_ref[...] + a1_ref[...]
    y = lax.dot_general(h, w_ref[...], (((1,), (1,)), ((), ())),
                        preferred_element_type=jnp.float32)
    o_ref[...] = jnp.maximum(y + b_ref[...], 0.0)


def _final(scale, x_pad, a0, a1, W, b2):
    grid = N_PAD // 1024
    return pl.pallas_call(
        _final_body,
        grid=(grid,),
        in_specs=[
            pl.BlockSpec(memory_space=pltpu.SMEM),
            pl.BlockSpec((1024, D), lambda i: (i, 0)),
            pl.BlockSpec((1024, D), lambda i: (i, 0)),
            pl.BlockSpec((1024, D), lambda i: (i, 0)),
            pl.BlockSpec((D, D), lambda i: (0, 0)),
            pl.BlockSpec((1, D), lambda i: (0, 0)),
        ],
        out_specs=pl.BlockSpec((1024, D), lambda i: (i, 0)),
        out_shape=jax.ShapeDtypeStruct((N_PAD, D), jnp.float32),
    )(scale, x_pad, a0, a1, W, b2)


def kernel(x, edge_index, W, b, eps):
    src = edge_index[0].astype(jnp.int32)
    dst = edge_index[1].astype(jnp.int32)
    n_fill = E_PAD - N_EDGES
    fill = jnp.arange(n_fill, dtype=jnp.int32)
    src_p = jnp.concatenate([src, fill % N_NODES]).reshape(NW, CH, B)
    dst_p = jnp.concatenate(
        [dst, N_NODES + fill % (N_PAD - N_NODES)]).reshape(NW, CH, B)

    x_pad = jnp.zeros((N_PAD, D), jnp.float32).at[:N_NODES].set(x)
    relux = _relu_table(x_pad)
    zeros = jnp.zeros((B, D), jnp.float32)
    agg2 = _sc_agg(relux, src_p, dst_p, zeros)

    scale = (1.0 + eps).astype(jnp.float32).reshape(1, 1)
    out = _final(scale, x_pad, agg2[0], agg2[1], W, b.reshape(1, D))
    return out[:N_NODES]


# speedup vs baseline: 12.0992x; 1.5205x over previous
import functools

import jax
import jax.numpy as jnp
from jax import lax
from jax.experimental import pallas as pl
from jax.experimental.pallas import tpu as pltpu
from jax.experimental.pallas import tpu_sc as plsc

N_NODES = 10000
N_PAD = 10240
D = 128
N_EDGES = 320000
NC = 2
NS = 16
NW = NC * NS
B = 128
CH = 80
E_PAD = NW * CH * B
ROWS_PER_TILE = N_PAD // NS
BM = 2000


def _relu_body(x_ref, o_ref):
    o_ref[...] = jnp.maximum(x_ref[...], 0.0)


def _relu_table(x):
    return pl.pallas_call(
        _relu_body,
        grid=(N_NODES // BM,),
        in_specs=[pl.BlockSpec((BM, D), lambda i: (i, 0))],
        out_specs=pl.BlockSpec((BM, D), lambda i: (i, 0)),
        out_shape=jax.ShapeDtypeStruct((N_PAD, D), jnp.float32),
    )(x)


def _sc_agg_body(relux_hbm, idx_hbm, zeros_hbm, out_hbm,
                 idx_v, rows0, rows1, src0, src1, dst0, dst1,
                 agg_sh, sem0, sem1):
    c = lax.axis_index("c")
    s = lax.axis_index("s")
    wid = s * NC + c

    for z in range(ROWS_PER_TILE // 128):
        pltpu.sync_copy(zeros_hbm,
                        agg_sh.at[pl.ds(s * ROWS_PER_TILE + z * 128, 128)])

    pltpu.sync_copy(idx_hbm.at[wid], idx_v)
    plsc.subcore_barrier()

    bufs = (rows0, rows1)
    sems = (sem0, sem1)
    srcs = (src0, src1)
    dsts = (dst0, dst1)

    def unpack(j, bsel):
        for k in range(B // 16):
            w = idx_v[j, pl.ds(k * 16, 16)]
            srcs[bsel][pl.ds(k * 16, 16)] = w & 0xFFFF
            dsts[bsel][pl.ds(k * 16, 16)] = lax.shift_right_logical(w, 16)

    for bsel in range(2):
        unpack(bsel, bsel)
        pltpu.async_copy(relux_hbm.at[srcs[bsel]], bufs[bsel], sems[bsel])

    def step(i, carry):
        j = i * 2
        for bsel in range(2):
            buf, sem = bufs[bsel], sems[bsel]
            jj = j + bsel
            pltpu.make_async_copy(relux_hbm.at[srcs[bsel]], buf, sem).wait()
            pltpu.sync_copy(buf, agg_sh.at[dsts[bsel]], add=True)

            @pl.when(jj + 2 < CH)
            def _():
                unpack(jj + 2, bsel)
                pltpu.async_copy(relux_hbm.at[srcs[bsel]], buf, sem)
        return carry

    lax.fori_loop(0, CH // 2, step, 0)
    plsc.subcore_barrier()

    r0 = s * ROWS_PER_TILE
    pltpu.sync_copy(agg_sh.at[pl.ds(r0, ROWS_PER_TILE)],
                    out_hbm.at[c].at[pl.ds(r0, ROWS_PER_TILE)])


_sc_agg = functools.partial(
    pl.kernel,
    out_type=jax.ShapeDtypeStruct((NC, N_PAD, D), jnp.float32),
    mesh=plsc.VectorSubcoreMesh(core_axis_name="c", subcore_axis_name="s"),
    scratch_types=[
        pltpu.VMEM((CH, B), jnp.int32),
        pltpu.VMEM((B, D), jnp.float32),
        pltpu.VMEM((B, D), jnp.float32),
        pltpu.VMEM((B,), jnp.int32),
        pltpu.VMEM((B,), jnp.int32),
        pltpu.VMEM((B,), jnp.int32),
        pltpu.VMEM((B,), jnp.int32),
        pltpu.VMEM_SHARED((N_PAD, D), jnp.float32),
        pltpu.SemaphoreType.DMA,
        pltpu.SemaphoreType.DMA,
    ],
)(_sc_agg_body)


def _final_body(scale_ref, x_ref, a0_ref, a1_ref, w_ref, b_ref, o_ref):
    h = scale_ref[0, 0] * x_ref[...] + a0_ref[0] + a1_ref[0]
    y = lax.dot_general(h, w_ref[...], (((1,), (1,)), ((), ())),
                        preferred_element_type=jnp.float32)
    o_ref[...] = jnp.maximum(y + b_ref[...], 0.0)


def _final(scale, x, agg2, W, b2):
    return pl.pallas_call(
        _final_body,
        grid=(N_NODES // BM,),
        in_specs=[
            pl.BlockSpec(memory_space=pltpu.SMEM),
            pl.BlockSpec((BM, D), lambda i: (i, 0)),
            pl.BlockSpec((1, BM, D), lambda i: (0, i, 0)),
            pl.BlockSpec((1, BM, D), lambda i: (1, i, 0)),
            pl.BlockSpec((D, D), lambda i: (0, 0)),
            pl.BlockSpec((1, D), lambda i: (0, 0)),
        ],
        out_specs=pl.BlockSpec((BM, D), lambda i: (i, 0)),
        out_shape=jax.ShapeDtypeStruct((N_NODES, D), jnp.float32),
    )(scale, x, agg2, agg2, W, b2)


def kernel(x, edge_index, W, b, eps):
    src = edge_index[0].astype(jnp.int32)
    dst = edge_index[1].astype(jnp.int32)
    n_fill = E_PAD - N_EDGES
    fill = jnp.arange(n_fill, dtype=jnp.int32)
    src_p = jnp.concatenate([src, fill % N_NODES])
    dst_p = jnp.concatenate([dst, N_NODES + fill % (N_PAD - N_NODES)])
    packed = (src_p | (dst_p << 16)).reshape(NW, CH, B)

    relux = _relu_table(x)
    zeros = jnp.zeros((128, D), jnp.float32)
    agg2 = _sc_agg(relux, packed, zeros)

    scale = (1.0 + eps).astype(jnp.float32).reshape(1, 1)
    return _final(scale, x, agg2, W, b.reshape(1, D))
